# initial kernel scaffold (unmeasured)
import jax
import jax.numpy as jnp
from jax import lax
from jax.experimental import pallas as pl
from jax.experimental.pallas import tpu as pltpu


def kernel(
    x,
):
    def body(*refs):
        pass

    out_shape = jax.ShapeDtypeStruct(..., jnp.float32)
    return pl.pallas_call(body, out_shape=out_shape)(...)


# baseline (device time: 32201 ns/iter reference)
import jax
import jax.numpy as jnp
from jax import lax
from jax.experimental import pallas as pl
from jax.experimental.pallas import tpu as pltpu


def kernel(x):
    m, n = x.shape
    half = m // 2

    def body(x_ref, out_ref, send_ref, recv_ref,
             send_sem1, recv_sem1, send_sem2, recv_sem2):
        my_x = lax.axis_index("x")
        my_y = lax.axis_index("y")
        x_nbr = (1 - my_x, my_y)
        y_nbr = (my_x, 1 - my_y)

        row0 = my_y * half
        send_ref[:, :] = x_ref[pl.ds(row0, half), :].astype(jnp.bfloat16)

        barrier_sem = pltpu.get_barrier_semaphore()
        for nbr in (x_nbr, y_nbr):
            pl.semaphore_signal(
                barrier_sem, inc=1,
                device_id=nbr, device_id_type=pl.DeviceIdType.MESH,
            )
        pl.semaphore_wait(barrier_sem, 2)

        rdma1 = pltpu.make_async_remote_copy(
            src_ref=send_ref,
            dst_ref=recv_ref,
            send_sem=send_sem1,
            recv_sem=recv_sem1,
            device_id=x_nbr,
            device_id_type=pl.DeviceIdType.MESH,
        )
        rdma1.start()
        rdma1.wait()

        out_ref[pl.ds(row0, half), :] = send_ref[:, :] + recv_ref[:, :]

        rdma2 = pltpu.make_async_remote_copy(
            src_ref=out_ref.at[pl.ds(row0, half), :],
            dst_ref=out_ref.at[pl.ds(row0, half), :],
            send_sem=send_sem2,
            recv_sem=recv_sem2,
            device_id=y_nbr,
            device_id_type=pl.DeviceIdType.MESH,
        )
        rdma2.start()
        rdma2.wait()

    return pl.pallas_call(
        body,
        out_shape=jax.ShapeDtypeStruct((m, n), jnp.bfloat16),
        in_specs=[pl.BlockSpec(memory_space=pltpu.VMEM)],
        out_specs=pl.BlockSpec(memory_space=pltpu.VMEM),
        scratch_shapes=[
            pltpu.VMEM((half, n), jnp.bfloat16),
            pltpu.VMEM((half, n), jnp.bfloat16),
            pltpu.SemaphoreType.DMA,
            pltpu.SemaphoreType.DMA,
            pltpu.SemaphoreType.DMA,
            pltpu.SemaphoreType.DMA,
        ],
        compiler_params=pltpu.CompilerParams(collective_id=0),
    )(x)


# device time: 22554 ns/iter; 1.4277x vs baseline; 1.4277x over previous
import jax
import jax.numpy as jnp
from jax import lax
from jax.experimental import pallas as pl
from jax.experimental.pallas import tpu as pltpu

N_CHUNKS = 8


def kernel(x):
    m, n = x.shape
    half = m // 2
    chunk = half // N_CHUNKS

    def body(x_ref, out_ref, send_ref, recv_ref,
             send_sems1, recv_sems1, send_sems2, recv_sems2):
        my_x = lax.axis_index("x")
        my_y = lax.axis_index("y")
        x_nbr = (1 - my_x, my_y)
        y_nbr = (my_x, 1 - my_y)

        row0 = my_y * half
        send_ref[:, :] = x_ref[pl.ds(row0, half), :].astype(jnp.bfloat16)

        barrier_sem = pltpu.get_barrier_semaphore()
        for nbr in (x_nbr, y_nbr):
            pl.semaphore_signal(
                barrier_sem, inc=1,
                device_id=nbr, device_id_type=pl.DeviceIdType.MESH,
            )
        pl.semaphore_wait(barrier_sem, 2)

        rdma1 = []
        for k in range(N_CHUNKS):
            r = pltpu.make_async_remote_copy(
                src_ref=send_ref.at[pl.ds(k * chunk, chunk), :],
                dst_ref=recv_ref.at[pl.ds(k * chunk, chunk), :],
                send_sem=send_sems1.at[k],
                recv_sem=recv_sems1.at[k],
                device_id=x_nbr,
                device_id_type=pl.DeviceIdType.MESH,
            )
            r.start()
            rdma1.append(r)

        rdma2 = []
        for k in range(N_CHUNKS):
            rdma1[k].wait_recv()
            rows = pl.ds(row0 + k * chunk, chunk)
            out_ref[rows, :] = (
                send_ref[pl.ds(k * chunk, chunk), :]
                + recv_ref[pl.ds(k * chunk, chunk), :]
            )
            r = pltpu.make_async_remote_copy(
                src_ref=out_ref.at[rows, :],
                dst_ref=out_ref.at[rows, :],
                send_sem=send_sems2.at[k],
                recv_sem=recv_sems2.at[k],
                device_id=y_nbr,
                device_id_type=pl.DeviceIdType.MESH,
            )
            r.start()
            rdma2.append(r)

        for k in range(N_CHUNKS):
            rdma1[k].wait_send()
            rdma2[k].wait()

    return pl.pallas_call(
        body,
        out_shape=jax.ShapeDtypeStruct((m, n), jnp.bfloat16),
        in_specs=[pl.BlockSpec(memory_space=pltpu.VMEM)],
        out_specs=pl.BlockSpec(memory_space=pltpu.VMEM),
        scratch_shapes=[
            pltpu.VMEM((half, n), jnp.bfloat16),
            pltpu.VMEM((half, n), jnp.bfloat16),
            pltpu.SemaphoreType.DMA((N_CHUNKS,)),
            pltpu.SemaphoreType.DMA((N_CHUNKS,)),
            pltpu.SemaphoreType.DMA((N_CHUNKS,)),
            pltpu.SemaphoreType.DMA((N_CHUNKS,)),
        ],
        compiler_params=pltpu.CompilerParams(collective_id=0),
    )(x)


# device time: 22145 ns/iter; 1.4541x vs baseline; 1.0185x over previous
import jax
import jax.numpy as jnp
from jax import lax
from jax.experimental import pallas as pl
from jax.experimental.pallas import tpu as pltpu

N_CHUNKS = 16


def kernel(x):
    m, n = x.shape
    half = m // 2
    chunk = half // N_CHUNKS

    def body(x_ref, out_ref, send_ref, recv_ref,
             send_sems1, recv_sems1, send_sems2, recv_sems2):
        my_x = lax.axis_index("x")
        my_y = lax.axis_index("y")
        x_nbr = (1 - my_x, my_y)
        y_nbr = (my_x, 1 - my_y)

        row0 = my_y * half
        send_ref[:, :] = x_ref[pl.ds(row0, half), :].astype(jnp.bfloat16)

        barrier_sem = pltpu.get_barrier_semaphore()
        for nbr in (x_nbr, y_nbr):
            pl.semaphore_signal(
                barrier_sem, inc=1,
                device_id=nbr, device_id_type=pl.DeviceIdType.MESH,
            )
        pl.semaphore_wait(barrier_sem, 2)

        rdma1 = []
        for k in range(N_CHUNKS):
            r = pltpu.make_async_remote_copy(
                src_ref=send_ref.at[pl.ds(k * chunk, chunk), :],
                dst_ref=recv_ref.at[pl.ds(k * chunk, chunk), :],
                send_sem=send_sems1.at[k],
                recv_sem=recv_sems1.at[k],
                device_id=x_nbr,
                device_id_type=pl.DeviceIdType.MESH,
            )
            r.start()
            rdma1.append(r)

        rdma2 = []
        for k in range(N_CHUNKS):
            rdma1[k].wait_recv()
            rows = pl.ds(row0 + k * chunk, chunk)
            out_ref[rows, :] = (
                send_ref[pl.ds(k * chunk, chunk), :]
                + recv_ref[pl.ds(k * chunk, chunk), :]
            )
            r = pltpu.make_async_remote_copy(
                src_ref=out_ref.at[rows, :],
                dst_ref=out_ref.at[rows, :],
                send_sem=send_sems2.at[k],
                recv_sem=recv_sems2.at[k],
                device_id=y_nbr,
                device_id_type=pl.DeviceIdType.MESH,
            )
            r.start()
            rdma2.append(r)

        for k in range(N_CHUNKS):
            rdma1[k].wait_send()
            rdma2[k].wait()

    return pl.pallas_call(
        body,
        out_shape=jax.ShapeDtypeStruct((m, n), jnp.bfloat16),
        in_specs=[pl.BlockSpec(memory_space=pltpu.VMEM)],
        out_specs=pl.BlockSpec(memory_space=pltpu.VMEM),
        scratch_shapes=[
            pltpu.VMEM((half, n), jnp.bfloat16),
            pltpu.VMEM((half, n), jnp.bfloat16),
            pltpu.SemaphoreType.DMA((N_CHUNKS,)),
            pltpu.SemaphoreType.DMA((N_CHUNKS,)),
            pltpu.SemaphoreType.DMA((N_CHUNKS,)),
            pltpu.SemaphoreType.DMA((N_CHUNKS,)),
        ],
        compiler_params=pltpu.CompilerParams(collective_id=0),
    )(x)


# device time: 19046 ns/iter; 1.6907x vs baseline; 1.1627x over previous
import os

import jax
import jax.numpy as jnp
from jax import lax
from jax.experimental import pallas as pl
from jax.experimental.pallas import tpu as pltpu

N_CHUNKS = 16

_DIAG = int(os.environ.get("KERNEL_DIAG", "0"))


def kernel(x):
    m, n = x.shape
    half = m // 2
    chunk = half // N_CHUNKS

    def body(x_ref, out_ref, send_ref, recv_ref,
             send_sems1, recv_sems1, send_sems2, recv_sems2):
        my_x = lax.axis_index("x")
        my_y = lax.axis_index("y")
        x_nbr = (1 - my_x, my_y)
        y_nbr = (my_x, 1 - my_y)

        row0 = my_y * half
        send_ref[:, :] = x_ref[pl.ds(row0, half), :].astype(jnp.bfloat16)

        if _DIAG == 2:
            for k in range(2 * N_CHUNKS):
                rows = pl.ds(k * chunk, chunk)
                out_ref[rows, :] = (
                    send_ref[pl.ds((k % N_CHUNKS) * chunk, chunk), :] * 2.0
                )
            return

        nbrs = (x_nbr,) if _DIAG == 1 else (x_nbr, y_nbr)
        barrier_sem = pltpu.get_barrier_semaphore()
        for nbr in nbrs:
            pl.semaphore_signal(
                barrier_sem, inc=1,
                device_id=nbr, device_id_type=pl.DeviceIdType.MESH,
            )
        pl.semaphore_wait(barrier_sem, len(nbrs))

        rdma1 = []
        for k in range(N_CHUNKS):
            r = pltpu.make_async_remote_copy(
                src_ref=send_ref.at[pl.ds(k * chunk, chunk), :],
                dst_ref=recv_ref.at[pl.ds(k * chunk, chunk), :],
                send_sem=send_sems1.at[k],
                recv_sem=recv_sems1.at[k],
                device_id=x_nbr,
                device_id_type=pl.DeviceIdType.MESH,
            )
            r.start()
            rdma1.append(r)

        rdma2 = []
        for k in range(N_CHUNKS):
            rdma1[k].wait_recv()
            rows = pl.ds(row0 + k * chunk, chunk)
            out_ref[rows, :] = (
                send_ref[pl.ds(k * chunk, chunk), :]
                + recv_ref[pl.ds(k * chunk, chunk), :]
            )
            if _DIAG == 0:
                r = pltpu.make_async_remote_copy(
                    src_ref=out_ref.at[rows, :],
                    dst_ref=out_ref.at[rows, :],
                    send_sem=send_sems2.at[k],
                    recv_sem=recv_sems2.at[k],
                    device_id=y_nbr,
                    device_id_type=pl.DeviceIdType.MESH,
                )
                r.start()
                rdma2.append(r)
            else:
                other = pl.ds((half - row0) + k * chunk, chunk)
                out_ref[other, :] = (
                    send_ref[pl.ds(k * chunk, chunk), :]
                    + recv_ref[pl.ds(k * chunk, chunk), :]
                )

        for k in range(N_CHUNKS):
            rdma1[k].wait_send()
            if _DIAG == 0:
                rdma2[k].wait()

    return pl.pallas_call(
        body,
        out_shape=jax.ShapeDtypeStruct((m, n), jnp.bfloat16),
        in_specs=[pl.BlockSpec(memory_space=pltpu.VMEM)],
        out_specs=pl.BlockSpec(memory_space=pltpu.VMEM),
        scratch_shapes=[
            pltpu.VMEM((half, n), jnp.bfloat16),
            pltpu.VMEM((half, n), jnp.bfloat16),
            pltpu.SemaphoreType.DMA((N_CHUNKS,)),
            pltpu.SemaphoreType.DMA((N_CHUNKS,)),
            pltpu.SemaphoreType.DMA((N_CHUNKS,)),
            pltpu.SemaphoreType.DMA((N_CHUNKS,)),
        ],
        compiler_params=pltpu.CompilerParams(
            collective_id=None if _DIAG == 2 else 0
        ),
    )(x)
